# Initial kernel scaffold; baseline (speedup 1.0000x reference)
import jax
import jax.numpy as jnp
from jax.experimental import pallas as pl


def kernel(h_user, h_post, user_context, edge_feat_comment, W_pub, b_pub, W_com, b_com, W_conv, b_conv, ln_g, ln_b, W_ep, b_ep, edge_index_publish, edge_index_comment, edge_index_ucu):
    raise NotImplementedError("write your pallas kernel here")


# speedup vs baseline: 1.7630x; 1.7630x over previous
import functools

import jax
import jax.numpy as jnp
from jax import lax
from jax.experimental import pallas as pl
from jax.experimental.pallas import tpu as pltpu
from jax.experimental.pallas import tpu_sc as plsc

N_NODE = 50000
E = 160000
D = 128
G = 32
NG = D // G
CONV = 16

NS = 16
K = 80
NBLK = E // K
BPW = NBLK // NS
EPW = E // NS
NPAD = 50048
RPW = NPAD // NS
ZR = 136

_f32 = jnp.float32


def _tables_body(hu, uc, wp, bp, wc, bc, w1, w2, bv, lg, lb, *outs):
    x = hu[...]
    tp = jnp.dot(x, wp[...], preferred_element_type=_f32) + bp[...]
    tc = (jnp.dot(x, wc[...], preferred_element_type=_f32) + bc[...]) * 0.7
    z = (jnp.dot(x, w1[...], preferred_element_type=_f32)
         + jnp.dot(uc[...], w2[...], preferred_element_type=_f32) + bv[...])
    mu = jnp.mean(z, axis=-1, keepdims=True)
    var = jnp.mean((z - mu) ** 2, axis=-1, keepdims=True)
    nm = jnp.maximum((z - mu) / jnp.sqrt(var + 1e-5) * lg[...] + lb[...], 0.0)
    for g in range(NG):
        outs[g][...] = tp[:, G * g:G * g + G]
        outs[NG + g][...] = tc[:, G * g:G * g + G]
        outs[2 * NG + g][...] = nm[:, G * g:G * g + G]


def _node_tables(h_user, user_context, W_pub, b_pub, W_com, b_com,
                 W1, W2, b_conv, ln_g, ln_b):
    blk = 1000
    grid = (N_NODE // blk,)
    full = lambda r, c: pl.BlockSpec((r, c), lambda i: (0, 0))
    return pl.pallas_call(
        _tables_body,
        grid=grid,
        in_specs=[
            pl.BlockSpec((blk, D), lambda i: (i, 0)),
            pl.BlockSpec((blk, CONV), lambda i: (i, 0)),
            full(D, D), full(1, D), full(D, D), full(1, D),
            full(D, D), full(CONV, D), full(1, D), full(1, D), full(1, D),
        ],
        out_specs=[pl.BlockSpec((blk, G), lambda i: (i, 0))] * (3 * NG),
        out_shape=[jax.ShapeDtypeStruct((N_NODE, G), _f32)] * (3 * NG),
    )(h_user, user_context, W_pub, b_pub.reshape(1, D), W_com,
      b_com.reshape(1, D), W1, W2, b_conv.reshape(1, D),
      ln_g.reshape(1, D), ln_b.reshape(1, D))


def _sc_body(*refs):
    (tbls, ef32, sp, dp, sc_, dc_, su, du,
     osum, acc, idxg, idxs, rows, cbuf, zbuf) = (
        refs[0:12], refs[12], refs[13], refs[14], refs[15], refs[16],
        refs[17], refs[18], refs[19:35], refs[35], refs[36], refs[37],
        refs[38], refs[39], refs[40])

    cid = lax.axis_index("c")
    sid = lax.axis_index("s")

    z16 = jnp.zeros((16,), _f32)
    one16 = jnp.where(lax.iota(jnp.int32, 16) == 0, 1.0, 0.0).astype(_f32)

    @pl.loop(0, ZR)
    def _(i):
        zbuf[i, pl.ds(0, 16)] = z16
        zbuf[i, pl.ds(16, 16)] = z16

    @pl.loop(0, K)
    def _(i):
        cbuf[i, pl.ds(0, 16)] = one16
        cbuf[i, pl.ds(16, 16)] = z16

    def one_pass(gather_tbl, seq_tbl, use_const, src_hbm, dst_hbm, out_hbm):
        for k in range(RPW // ZR):
            pltpu.sync_copy(zbuf, acc.at[pl.ds(sid * RPW + k * ZR, ZR)])
        plsc.subcore_barrier()
        pltpu.sync_copy(dst_hbm.at[sid], idxs)
        if gather_tbl is not None:
            pltpu.sync_copy(src_hbm.at[sid], idxg)

        @pl.loop(0, BPW)
        def _(j):
            if gather_tbl is not None:
                pltpu.sync_copy(gather_tbl.at[idxg.at[j]], rows)
                src_buf = rows
            elif seq_tbl is not None:
                pltpu.sync_copy(seq_tbl.at[pl.ds(sid * EPW + j * K, K)], rows)
                src_buf = rows
            else:
                src_buf = cbuf
            pltpu.sync_copy(src_buf, acc.at[idxs.at[j]], add=True)

        plsc.subcore_barrier()
        pltpu.sync_copy(acc.at[pl.ds(sid * RPW, RPW)],
                        out_hbm.at[pl.ds(sid * RPW, RPW)])
        plsc.subcore_barrier()

    passes = []
    for g in range(NG):
        owner = 0 if g < 2 else 1
        passes.append((tbls[g], None, False, sp, dp, osum[g], owner))
        passes.append((tbls[NG + g], None, False, sc_, dc_, osum[NG + g], owner))
        passes.append((tbls[2 * NG + g], None, False, su, du, osum[2 * NG + g], owner))
    passes.append((None, ef32, False, None, dc_, osum[12], 0))
    passes.append((None, None, True, None, dp, osum[13], 1))
    passes.append((None, None, True, None, dc_, osum[14], 1))
    passes.append((None, None, True, None, du, osum[15], 0))

    for gt, st, uc_, s_h, d_h, o_h, owner in passes:
        @pl.when(cid == owner)
        def _(gt=gt, st=st, s_h=s_h, d_h=d_h, o_h=o_h):
            one_pass(gt, st, uc_, s_h, d_h, o_h)


def _sc_segment_sums(tables, ef32, sp, dp, sc_, dc_, su, du):
    mesh = plsc.VectorSubcoreMesh(core_axis_name="c", subcore_axis_name="s")
    kern = pl.kernel(
        _sc_body,
        out_type=[jax.ShapeDtypeStruct((NPAD, G), _f32)] * 16,
        mesh=mesh,
        compiler_params=pltpu.CompilerParams(use_tc_tiling_on_sc=False),
        scratch_types=[
            pltpu.VMEM_SHARED((NPAD, G), _f32),
            pltpu.VMEM((BPW, K), jnp.int32),
            pltpu.VMEM((BPW, K), jnp.int32),
            pltpu.VMEM((K, G), _f32),
            pltpu.VMEM((K, G), _f32),
            pltpu.VMEM((ZR, G), _f32),
        ],
    )
    return kern(*tables, ef32, sp, dp, sc_, dc_, su, du)


def _fin_body(wep, bep, *refs):
    sums = refs[0:12]
    efs, cp, cc, cu = refs[12], refs[13], refs[14], refs[15]
    pub, com, ucu = refs[16], refs[17], refs[18]
    cnt_c = cc[...][:, 0:1]
    invp = 1.0 / jnp.maximum(cp[...][:, 0:1], 1.0)
    invc = 1.0 / jnp.maximum(cnt_c, 1.0)
    invu = 1.0 / jnp.maximum(cu[...][:, 0:1], 1.0)
    pub[...] = jnp.concatenate([sums[g][...] for g in range(NG)], axis=1) * invp
    nonzero = jnp.where(cnt_c >= 1.0, 0.3, 0.0)
    base = (jnp.dot(efs[...][:, 0:CONV] * invc, wep[...],
                    preferred_element_type=_f32) + bep[...]) * nonzero
    com[...] = jnp.concatenate([sums[NG + g][...] for g in range(NG)],
                               axis=1) * invc + base
    ucu[...] = jnp.concatenate([sums[2 * NG + g][...] for g in range(NG)],
                               axis=1) * invu


def _finalize(W_ep, b_ep, sums):
    blk = 1000
    grid = (N_NODE // blk,)
    return pl.pallas_call(
        _fin_body,
        grid=grid,
        in_specs=[pl.BlockSpec((CONV, D), lambda i: (0, 0)),
                  pl.BlockSpec((1, D), lambda i: (0, 0))] +
                 [pl.BlockSpec((blk, G), lambda i: (i, 0))] * 16,
        out_specs=[pl.BlockSpec((blk, D), lambda i: (i, 0))] * 3,
        out_shape=[jax.ShapeDtypeStruct((N_NODE, D), _f32)] * 3,
    )(W_ep, b_ep.reshape(1, D), *sums)


@jax.jit
def kernel(h_user, h_post, user_context, edge_feat_comment, W_pub, b_pub,
           W_com, b_com, W_conv, b_conv, ln_g, ln_b, W_ep, b_ep,
           edge_index_publish, edge_index_comment, edge_index_ucu):
    tables = _node_tables(h_user, user_context, W_pub, b_pub, W_com, b_com,
                          W_conv[:D], W_conv[D:], b_conv, ln_g, ln_b)
    ef32 = jnp.concatenate(
        [edge_feat_comment, jnp.zeros((E, G - CONV), _f32)], axis=1)
    i32 = jnp.int32
    sp = edge_index_publish[0].astype(i32).reshape(NS, BPW, K)
    dp = edge_index_publish[1].astype(i32).reshape(NS, BPW, K)
    sc_ = edge_index_comment[0].astype(i32).reshape(NS, BPW, K)
    dc_ = edge_index_comment[1].astype(i32).reshape(NS, BPW, K)
    su = edge_index_ucu[0].astype(i32).reshape(NS, BPW, K)
    du = edge_index_ucu[1].astype(i32).reshape(NS, BPW, K)
    sums = _sc_segment_sums(tables, ef32, sp, dp, sc_, dc_, su, du)
    pub, com, ucu = _finalize(W_ep, b_ep, sums)
    return (pub, com, ucu)


# speedup vs baseline: 2.2823x; 1.2946x over previous
import functools

import jax
import jax.numpy as jnp
from jax import lax
from jax.experimental import pallas as pl
from jax.experimental.pallas import tpu as pltpu
from jax.experimental.pallas import tpu_sc as plsc

N_NODE = 50000
E = 160000
D = 128
G = 32
NG = D // G
CONV = 16

NS = 16
K = 80
NBLK = E // K
BPW = NBLK // NS
EPW = E // NS
NPAD = 50048
RPW = NPAD // NS

_f32 = jnp.float32


def _tables_body(hu, uc, wp, bp, wc, bc, w1, w2, bv, lg, lb, *outs):
    x = hu[...]
    tp = jnp.dot(x, wp[...], preferred_element_type=_f32) + bp[...]
    tc = (jnp.dot(x, wc[...], preferred_element_type=_f32) + bc[...]) * 0.7
    z = (jnp.dot(x, w1[...], preferred_element_type=_f32)
         + jnp.dot(uc[...], w2[...], preferred_element_type=_f32) + bv[...])
    mu = jnp.mean(z, axis=-1, keepdims=True)
    var = jnp.mean((z - mu) ** 2, axis=-1, keepdims=True)
    nm = jnp.maximum((z - mu) / jnp.sqrt(var + 1e-5) * lg[...] + lb[...], 0.0)
    for g in range(NG):
        outs[g][...] = tp[:, G * g:G * g + G]
        outs[NG + g][...] = tc[:, G * g:G * g + G]
        outs[2 * NG + g][...] = nm[:, G * g:G * g + G]


def _node_tables(h_user, user_context, W_pub, b_pub, W_com, b_com,
                 W1, W2, b_conv, ln_g, ln_b):
    blk = 1000
    grid = (N_NODE // blk,)
    full = lambda r, c: pl.BlockSpec((r, c), lambda i: (0, 0))
    return pl.pallas_call(
        _tables_body,
        grid=grid,
        in_specs=[
            pl.BlockSpec((blk, D), lambda i: (i, 0)),
            pl.BlockSpec((blk, CONV), lambda i: (i, 0)),
            full(D, D), full(1, D), full(D, D), full(1, D),
            full(D, D), full(CONV, D), full(1, D), full(1, D), full(1, D),
        ],
        out_specs=[pl.BlockSpec((blk, G), lambda i: (i, 0))] * (3 * NG),
        out_shape=[jax.ShapeDtypeStruct((N_NODE, G), _f32)] * (3 * NG),
    )(h_user, user_context, W_pub, b_pub.reshape(1, D), W_com,
      b_com.reshape(1, D), W1, W2, b_conv.reshape(1, D),
      ln_g.reshape(1, D), ln_b.reshape(1, D))


def _sc_body(*refs):
    (tbls, ef32, sp, dp, sc_, dc_, su, du, osum,
     acc, idxg, idxs, r0, r1, r2, cbuf, g0, g1, g2, ss) = (
        refs[0:12], refs[12], refs[13], refs[14], refs[15], refs[16],
        refs[17], refs[18], refs[19:35], refs[35], refs[36], refs[37],
        refs[38], refs[39], refs[40], refs[41], refs[42], refs[43],
        refs[44], refs[45])

    cid = lax.axis_index("c")
    sid = lax.axis_index("s")
    bufs = (r0, r1, r2)
    gsems = (g0, g1, g2)

    z16 = jnp.zeros((16,), _f32)
    one16 = jnp.where(lax.iota(jnp.int32, 16) == 0, 1.0, 0.0).astype(_f32)

    @pl.loop(0, K)
    def _(i):
        cbuf[i, pl.ds(0, 16)] = one16
        cbuf[i, pl.ds(16, 16)] = z16

    def one_pass(gather_tbl, seq_tbl, src_hbm, dst_hbm, out_hbm):
        @pl.loop(0, K)
        def _(i):
            r0[i, pl.ds(0, 16)] = z16
            r0[i, pl.ds(16, 16)] = z16
        base = sid * RPW
        for k in range(RPW // K):
            pltpu.sync_copy(r0, acc.at[pl.ds(base + k * K, K)])
        pltpu.sync_copy(r0.at[pl.ds(0, RPW % K)],
                        acc.at[pl.ds(base + (RPW // K) * K, RPW % K)])
        plsc.subcore_barrier()

        pltpu.sync_copy(dst_hbm.at[sid], idxs)
        if gather_tbl is not None:
            pltpu.sync_copy(src_hbm.at[sid], idxg)

        if gather_tbl is None and seq_tbl is None:
            @pl.loop(0, BPW)
            def _(j):
                pltpu.async_copy(cbuf, acc.at[idxs.at[j]], ss, add=True)

            @pl.loop(0, BPW)
            def _(j):
                pltpu.make_async_copy(cbuf, acc.at[idxs.at[j]], ss).wait()
        else:
            def g_desc(j, b):
                if gather_tbl is not None:
                    return (gather_tbl.at[idxg.at[j]], bufs[b], gsems[b])
                return (seq_tbl.at[pl.ds(sid * EPW + j * K, K)],
                        bufs[b], gsems[b])

            pltpu.async_copy(*g_desc(0, 0))
            pltpu.async_copy(*g_desc(1, 1))

            @pl.loop(0, BPW, step=3)
            def _(j0):
                for b in range(3):
                    j = j0 + b

                    @pl.when(j < BPW)
                    def _(j=j, b=b):
                        pltpu.make_async_copy(*g_desc(j, b)).wait()

                        @pl.when(j + 2 < BPW)
                        def _(j=j, b=b):
                            pltpu.async_copy(*g_desc(j + 2, (b + 2) % 3))
                        pltpu.sync_copy(bufs[b], acc.at[idxs.at[j]], add=True)

        plsc.subcore_barrier()
        pltpu.sync_copy(acc.at[pl.ds(sid * RPW, RPW)],
                        out_hbm.at[pl.ds(sid * RPW, RPW)])
        plsc.subcore_barrier()

    passes = []
    for g in range(NG):
        owner = 0 if g < 2 else 1
        passes.append((tbls[g], None, sp, dp, osum[g], owner))
        passes.append((tbls[NG + g], None, sc_, dc_, osum[NG + g], owner))
        passes.append((tbls[2 * NG + g], None, su, du, osum[2 * NG + g], owner))
    passes.append((None, ef32, None, dc_, osum[12], 0))
    passes.append((None, None, None, dp, osum[13], 1))
    passes.append((None, None, None, dc_, osum[14], 1))
    passes.append((None, None, None, du, osum[15], 0))

    for gt, st, s_h, d_h, o_h, owner in passes:
        @pl.when(cid == owner)
        def _(gt=gt, st=st, s_h=s_h, d_h=d_h, o_h=o_h):
            one_pass(gt, st, s_h, d_h, o_h)


def _sc_segment_sums(tables, ef32, sp, dp, sc_, dc_, su, du):
    mesh = plsc.VectorSubcoreMesh(core_axis_name="c", subcore_axis_name="s")
    kern = pl.kernel(
        _sc_body,
        out_type=[jax.ShapeDtypeStruct((NPAD, G), _f32)] * 16,
        mesh=mesh,
        compiler_params=pltpu.CompilerParams(use_tc_tiling_on_sc=False),
        scratch_types=[
            pltpu.VMEM_SHARED((NPAD, G), _f32),
            pltpu.VMEM((BPW, K), jnp.int32),
            pltpu.VMEM((BPW, K), jnp.int32),
            pltpu.VMEM((K, G), _f32),
            pltpu.VMEM((K, G), _f32),
            pltpu.VMEM((K, G), _f32),
            pltpu.VMEM((K, G), _f32),
            pltpu.SemaphoreType.DMA,
            pltpu.SemaphoreType.DMA,
            pltpu.SemaphoreType.DMA,
            pltpu.SemaphoreType.DMA,
        ],
    )
    return kern(*tables, ef32, sp, dp, sc_, dc_, su, du)


def _fin_body(wep, bep, *refs):
    sums = refs[0:12]
    efs, cp, cc, cu = refs[12], refs[13], refs[14], refs[15]
    pub, com, ucu = refs[16], refs[17], refs[18]
    cnt_c = cc[...][:, 0:1]
    invp = 1.0 / jnp.maximum(cp[...][:, 0:1], 1.0)
    invc = 1.0 / jnp.maximum(cnt_c, 1.0)
    invu = 1.0 / jnp.maximum(cu[...][:, 0:1], 1.0)
    pub[...] = jnp.concatenate([sums[g][...] for g in range(NG)], axis=1) * invp
    nonzero = jnp.where(cnt_c >= 1.0, 0.3, 0.0)
    base = (jnp.dot(efs[...][:, 0:CONV] * invc, wep[...],
                    preferred_element_type=_f32) + bep[...]) * nonzero
    com[...] = jnp.concatenate([sums[NG + g][...] for g in range(NG)],
                               axis=1) * invc + base
    ucu[...] = jnp.concatenate([sums[2 * NG + g][...] for g in range(NG)],
                               axis=1) * invu


def _finalize(W_ep, b_ep, sums):
    blk = 1000
    grid = (N_NODE // blk,)
    return pl.pallas_call(
        _fin_body,
        grid=grid,
        in_specs=[pl.BlockSpec((CONV, D), lambda i: (0, 0)),
                  pl.BlockSpec((1, D), lambda i: (0, 0))] +
                 [pl.BlockSpec((blk, G), lambda i: (i, 0))] * 16,
        out_specs=[pl.BlockSpec((blk, D), lambda i: (i, 0))] * 3,
        out_shape=[jax.ShapeDtypeStruct((N_NODE, D), _f32)] * 3,
    )(W_ep, b_ep.reshape(1, D), *sums)


@jax.jit
def kernel(h_user, h_post, user_context, edge_feat_comment, W_pub, b_pub,
           W_com, b_com, W_conv, b_conv, ln_g, ln_b, W_ep, b_ep,
           edge_index_publish, edge_index_comment, edge_index_ucu):
    tables = _node_tables(h_user, user_context, W_pub, b_pub, W_com, b_com,
                          W_conv[:D], W_conv[D:], b_conv, ln_g, ln_b)
    ef32 = jnp.concatenate(
        [edge_feat_comment, jnp.zeros((E, G - CONV), _f32)], axis=1)
    i32 = jnp.int32
    sp = edge_index_publish[0].astype(i32).reshape(NS, BPW, K)
    dp = edge_index_publish[1].astype(i32).reshape(NS, BPW, K)
    sc_ = edge_index_comment[0].astype(i32).reshape(NS, BPW, K)
    dc_ = edge_index_comment[1].astype(i32).reshape(NS, BPW, K)
    su = edge_index_ucu[0].astype(i32).reshape(NS, BPW, K)
    du = edge_index_ucu[1].astype(i32).reshape(NS, BPW, K)
    sums = _sc_segment_sums(tables, ef32, sp, dp, sc_, dc_, su, du)
    pub, com, ucu = _finalize(W_ep, b_ep, sums)
    return (pub, com, ucu)
